# baseline (speedup 1.0000x reference)
import functools

import jax
import jax.numpy as jnp
from jax import lax
from jax.experimental import pallas as pl
from jax.experimental.pallas import tpu as pltpu
from jax.experimental.pallas import tpu_sc as plsc

V = 784
D = 2048
E = 8
EP = 128
RH = 64
NC = 10
S = 2048
NB = 4
N = NB * S
CH = 16
CL = 49


def _router_body(w1_ref, b1_ref, w2_ref, b2_ref, out_ref, wt_ref):
    t = lax.broadcasted_iota(jnp.int32, (V, 1), 0).astype(jnp.float32) / 783.0
    h = t * w1_ref[...] + b1_ref[...]
    h = h * jax.nn.sigmoid(h)
    rlog = jnp.dot(h, w2_ref[...],
                   preferred_element_type=jnp.float32) + b2_ref[...]
    mx = jnp.max(rlog, axis=1, keepdims=True)
    eg = jnp.exp(rlog - mx)
    gates = eg / jnp.sum(eg, axis=1, keepdims=True)
    idx = lax.broadcasted_iota(jnp.int32, (V, EP), 1)
    m1 = jnp.max(gates, axis=1, keepdims=True)
    i1 = jnp.min(jnp.where(gates == m1, idx, EP), axis=1, keepdims=True)
    g2 = jnp.where(idx == i1, -1.0, gates)
    m2 = jnp.max(g2, axis=1, keepdims=True)
    i2 = jnp.min(jnp.where(g2 == m2, idx, EP), axis=1, keepdims=True)
    masks = jnp.logical_or(idx == i1, idx == i2).astype(jnp.float32)
    weights = gates * masks / (m1 + m2 + 1e-9)
    out_ref[...] = jnp.concatenate(
        [weights[:, :E], masks[:, :E],
         jnp.zeros((V, EP - 2 * E), jnp.float32)], axis=1)
    wt_ref[...] = jnp.transpose(weights[:, :E], (1, 0))


def _router_table(router_w1, router_b1, router_w2, router_b2):
    w1 = router_w1.reshape(1, RH)
    b1 = router_b1.reshape(1, RH)
    w2 = jnp.zeros((RH, EP), jnp.float32).at[:, :E].set(router_w2)
    b2 = jnp.full((1, EP), -1e30, jnp.float32).at[0, :E].set(router_b2)
    return pl.pallas_call(
        _router_body,
        out_shape=[jax.ShapeDtypeStruct((V, EP), jnp.float32),
                   jax.ShapeDtypeStruct((E, V), jnp.float32)],
    )(w1, b1, w2, b2)


def _dense_body(wt_ref, ew_ref, eb_ref, cnt_ref, cw_ref, cb_ref, out_ref,
                st_ref):
    a = ew_ref[...] * (1.0 / 783.0)
    c2b = jnp.broadcast_to(
        (2.0 * jnp.cos(CH * a))[:, None, :], (E, CH, D))
    base = lax.broadcasted_iota(jnp.int32, (1, CH, 1), 1).astype(jnp.float32)
    ang0 = base * a[:, None, :] + eb_ref[...][:, None, :]
    s0 = jnp.sin(ang0)
    s1 = jnp.sin(ang0 + CH * a[:, None, :])

    st_ref[0:1] = s0.astype(jnp.bfloat16).reshape(1, E * CH, D)
    st_ref[1:2] = s1.astype(jnp.bfloat16).reshape(1, E * CH, D)

    def body(j, carry):
        sp, sc = carry
        cur = c2b * sc - sp
        st_ref[pl.ds(j, 1)] = cur.astype(jnp.bfloat16).reshape(1, E * CH, D)
        return (sc, cur)

    lax.fori_loop(2, CL, body, (s0, s1))

    cnt = cnt_ref[:, :V]
    pooled = jnp.zeros((NB, D), jnp.float32)
    for e in range(E):
        lhs = (cnt * wt_ref[e:e + 1, :]).astype(jnp.bfloat16)
        se = st_ref[:, e * CH:(e + 1) * CH, :].reshape(V, D)
        pooled = pooled + jnp.dot(lhs, se,
                                  preferred_element_type=jnp.float32)
    out_ref[...] = jnp.dot(pooled * (1.0 / S), cw_ref[...],
                           preferred_element_type=jnp.float32) + cb_ref[...]


def _dense_logits(wt, expert_w, expert_b, cnt, cls_w, cls_b):
    cw = jnp.zeros((D, EP), jnp.float32).at[:, :NC].set(cls_w)
    cb = jnp.zeros((1, EP), jnp.float32).at[0, :NC].set(cls_b)
    return pl.pallas_call(
        _dense_body,
        out_shape=jax.ShapeDtypeStruct((NB, EP), jnp.float32),
        scratch_shapes=[pltpu.VMEM((CL, E * CH, D), jnp.bfloat16)],
    )(wt, expert_w, expert_b, cnt, cw, cb)


VP = 800
NBVP = NB * VP


def _gather_tokens(wm_table, idx2d, idxb2d, zeros, ones):
    info = plsc.get_sparse_core_info()
    nw = info.num_cores * info.num_subcores
    b_per_w = N // nw
    ch = 128
    nch = b_per_w // ch
    mesh = plsc.VectorSubcoreMesh(core_axis_name="c", subcore_axis_name="s")

    @functools.partial(
        pl.kernel, mesh=mesh,
        out_type=[jax.ShapeDtypeStruct((N, EP), jnp.float32),
                  jax.ShapeDtypeStruct((info.num_cores, NBVP), jnp.float32)],
        scratch_types=[pltpu.VMEM((nch, ch), jnp.int32),
                       pltpu.VMEM((nch, ch), jnp.int32),
                       pltpu.VMEM((b_per_w, EP), jnp.float32),
                       pltpu.VMEM((ch,), jnp.float32),
                       pltpu.VMEM_SHARED((NBVP,), jnp.float32),
                       pltpu.SemaphoreType.DMA],
    )
    def k(table_hbm, idx_hbm, idxb_hbm, zeros_hbm, ones_hbm, out_hbm,
          hist_hbm, idx_v, idxb_v, rows_v, ones_v, shared, sem):
        s = lax.axis_index("s")
        c = lax.axis_index("c")
        wid = s * info.num_cores + c
        pltpu.sync_copy(idx_hbm.at[pl.ds(wid * nch, nch)], idx_v)
        pltpu.sync_copy(idxb_hbm.at[pl.ds(wid * nch, nch)], idxb_v)
        pltpu.sync_copy(ones_hbm, ones_v)

        @pl.when(s == 0)
        def _():
            pltpu.sync_copy(zeros_hbm, shared)

        plsc.subcore_barrier()
        descs = [pltpu.async_copy(table_hbm.at[idx_v.at[j]],
                                  rows_v.at[pl.ds(j * ch, ch)], sem)
                 for j in range(nch)]
        for j in range(nch):
            pltpu.sync_copy(ones_v, shared.at[idxb_v.at[j]], add=True)
        for d in descs:
            d.wait()
        pltpu.sync_copy(rows_v, out_hbm.at[pl.ds(wid * b_per_w, b_per_w)])
        plsc.subcore_barrier()

        @pl.when(s == 0)
        def _():
            pltpu.sync_copy(shared, hist_hbm.at[c])

    return k(wm_table, idx2d, idxb2d, zeros, ones)


def kernel(x, expert_w, expert_b, router_w1, router_b1, router_w2,
           router_b2, cls_w, cls_b):
    xi = x.astype(jnp.int32)
    xb = xi + (jnp.arange(NB, dtype=jnp.int32) * VP)[:, None]
    tab, wt = _router_table(router_w1, router_b1, router_w2, router_b2)
    wm, hist = _gather_tokens(
        tab, xi.reshape(-1, 128), xb.reshape(-1, 128),
        jnp.zeros((NBVP,), jnp.float32), jnp.ones((128,), jnp.float32))
    cnt = (hist[0] + hist[1]).reshape(NB, VP)
    logits = _dense_logits(wt, expert_w, expert_b, cnt, cls_w, cls_b)
    return logits[:, :NC], wm[:, :E], wm[:, E:2 * E]


# speedup vs baseline: 1.0592x; 1.0592x over previous
import functools

import jax
import jax.numpy as jnp
from jax import lax
from jax.experimental import pallas as pl
from jax.experimental.pallas import tpu as pltpu
from jax.experimental.pallas import tpu_sc as plsc

V = 784
D = 2048
E = 8
EP = 128
RH = 64
NC = 10
S = 2048
NB = 4
N = NB * S
CH = 16
CL = 49
LP = 1024


def _router_body(w1_ref, b1_ref, w2_ref, b2_ref, out_ref, wt_ref):
    t = lax.broadcasted_iota(jnp.int32, (V, 1), 0).astype(jnp.float32) / 783.0
    h = t * w1_ref[...] + b1_ref[...]
    h = h * jax.nn.sigmoid(h)
    rlog = jnp.dot(h, w2_ref[...],
                   preferred_element_type=jnp.float32) + b2_ref[...]
    mx = jnp.max(rlog, axis=1, keepdims=True)
    eg = jnp.exp(rlog - mx)
    gates = eg / jnp.sum(eg, axis=1, keepdims=True)
    idx = lax.broadcasted_iota(jnp.int32, (V, EP), 1)
    m1 = jnp.max(gates, axis=1, keepdims=True)
    i1 = jnp.min(jnp.where(gates == m1, idx, EP), axis=1, keepdims=True)
    g2 = jnp.where(idx == i1, -1.0, gates)
    m2 = jnp.max(g2, axis=1, keepdims=True)
    i2 = jnp.min(jnp.where(g2 == m2, idx, EP), axis=1, keepdims=True)
    masks = jnp.logical_or(idx == i1, idx == i2).astype(jnp.float32)
    weights = gates * masks / (m1 + m2 + 1e-9)
    out_ref[...] = jnp.concatenate(
        [weights[:, :E], masks[:, :E],
         jnp.zeros((V, EP - 2 * E), jnp.float32)], axis=1)
    wt_ref[...] = jnp.transpose(weights[:, :E], (1, 0))


def _router_table(router_w1, router_b1, router_w2, router_b2):
    w1 = router_w1.reshape(1, RH)
    b1 = router_b1.reshape(1, RH)
    w2 = jnp.zeros((RH, EP), jnp.float32).at[:, :E].set(router_w2)
    b2 = jnp.full((1, EP), -1e30, jnp.float32).at[0, :E].set(router_b2)
    return pl.pallas_call(
        _router_body,
        out_shape=[jax.ShapeDtypeStruct((V, EP), jnp.float32),
                   jax.ShapeDtypeStruct((E, V), jnp.float32)],
    )(w1, b1, w2, b2)


def _dense_body(xt_ref, wt_ref, ew_ref, eb_ref, cw_ref, cb_ref, out_ref,
                st_ref):
    a = ew_ref[...] * (1.0 / 783.0)
    c2b = jnp.broadcast_to(
        (2.0 * jnp.cos(CH * a))[:, None, :], (E, CH, D))
    base = lax.broadcasted_iota(jnp.int32, (1, CH, 1), 1).astype(jnp.float32)
    ang0 = base * a[:, None, :] + eb_ref[...][:, None, :]
    s0 = jnp.sin(ang0)
    s1 = jnp.sin(ang0 + CH * a[:, None, :])

    st_ref[0:1] = s0.astype(jnp.bfloat16).reshape(1, E * CH, D)
    st_ref[1:2] = s1.astype(jnp.bfloat16).reshape(1, E * CH, D)

    def body(j, carry):
        sp, sc = carry
        cur = c2b * sc - sp
        st_ref[pl.ds(j, 1)] = cur.astype(jnp.bfloat16).reshape(1, E * CH, D)
        return (sc, cur)

    lax.fori_loop(2, CL, body, (s0, s1))

    lane = lax.broadcasted_iota(jnp.int32, (1, LP), 1)
    ones_row = jnp.ones((1, S), jnp.bfloat16)
    rows = []
    for b in range(NB):
        oh = (xt_ref[:, b:b + 1] == lane).astype(jnp.bfloat16)
        rows.append(jnp.dot(ones_row, oh,
                            preferred_element_type=jnp.float32))
    cnt = jnp.concatenate(rows, axis=0)[:, :V]
    pooled = jnp.zeros((NB, D), jnp.float32)
    for e in range(E):
        lhs = (cnt * wt_ref[e:e + 1, :]).astype(jnp.bfloat16)
        se = st_ref[:, e * CH:(e + 1) * CH, :].reshape(V, D)
        pooled = pooled + jnp.dot(lhs, se,
                                  preferred_element_type=jnp.float32)
    out_ref[...] = jnp.dot(pooled * (1.0 / S), cw_ref[...],
                           preferred_element_type=jnp.float32) + cb_ref[...]


def _dense_logits(xt, wt, expert_w, expert_b, cls_w, cls_b):
    cw = jnp.zeros((D, EP), jnp.float32).at[:, :NC].set(cls_w)
    cb = jnp.zeros((1, EP), jnp.float32).at[0, :NC].set(cls_b)
    return pl.pallas_call(
        _dense_body,
        out_shape=jax.ShapeDtypeStruct((NB, EP), jnp.float32),
        scratch_shapes=[pltpu.VMEM((CL, E * CH, D), jnp.bfloat16)],
    )(xt, wt, expert_w, expert_b, cw, cb)


def _gather_tokens(wm_table, idx2d):
    info = plsc.get_sparse_core_info()
    nw = info.num_cores * info.num_subcores
    b_per_w = N // nw
    ch = 128
    nch = b_per_w // ch
    mesh = plsc.VectorSubcoreMesh(core_axis_name="c", subcore_axis_name="s")

    @functools.partial(
        pl.kernel, mesh=mesh,
        out_type=jax.ShapeDtypeStruct((N, EP), jnp.float32),
        scratch_types=[pltpu.VMEM((nch, ch), jnp.int32),
                       pltpu.VMEM((b_per_w, EP), jnp.float32),
                       pltpu.SemaphoreType.DMA],
    )
    def k(table_hbm, idx_hbm, out_hbm, idx_v, rows_v, sem):
        s = lax.axis_index("s")
        c = lax.axis_index("c")
        wid = s * info.num_cores + c
        pltpu.sync_copy(idx_hbm.at[pl.ds(wid * nch, nch)], idx_v)
        descs = [pltpu.async_copy(table_hbm.at[idx_v.at[j]],
                                  rows_v.at[pl.ds(j * ch, ch)], sem)
                 for j in range(nch)]
        for d in descs:
            d.wait()
        pltpu.sync_copy(rows_v, out_hbm.at[pl.ds(wid * b_per_w, b_per_w)])

    return k(wm_table, idx2d)


def kernel(x, expert_w, expert_b, router_w1, router_b1, router_w2,
           router_b2, cls_w, cls_b):
    xi = x.astype(jnp.int32)
    tab, wt = _router_table(router_w1, router_b1, router_w2, router_b2)
    wm = _gather_tokens(tab, xi.reshape(-1, 128))
    logits = _dense_logits(xi.T, wt, expert_w, expert_b, cls_w, cls_b)
    return logits[:, :NC], wm[:, :E], wm[:, E:2 * E]


# speedup vs baseline: 1.1176x; 1.0551x over previous
import functools

import jax
import jax.numpy as jnp
from jax import lax
from jax.experimental import pallas as pl
from jax.experimental.pallas import tpu as pltpu
from jax.experimental.pallas import tpu_sc as plsc

V = 784
D = 2048
E = 8
EP = 128
RH = 64
NC = 10
S = 2048
NB = 4
N = NB * S
CH = 16
CL = 49
LP = 1024


def _router_body(w1_ref, b1_ref, w2_ref, b2_ref, out_ref, wt_ref):
    t = lax.broadcasted_iota(jnp.int32, (V, 1), 0).astype(jnp.float32) / 783.0
    h = t * w1_ref[...] + b1_ref[...]
    h = h * jax.nn.sigmoid(h)
    rlog = jnp.dot(h, w2_ref[...],
                   preferred_element_type=jnp.float32) + b2_ref[...]
    mx = jnp.max(rlog, axis=1, keepdims=True)
    eg = jnp.exp(rlog - mx)
    gates = eg / jnp.sum(eg, axis=1, keepdims=True)
    idx = lax.broadcasted_iota(jnp.int32, (V, EP), 1)
    m1 = jnp.max(gates, axis=1, keepdims=True)
    i1 = jnp.min(jnp.where(gates == m1, idx, EP), axis=1, keepdims=True)
    g2 = jnp.where(idx == i1, -1.0, gates)
    m2 = jnp.max(g2, axis=1, keepdims=True)
    i2 = jnp.min(jnp.where(g2 == m2, idx, EP), axis=1, keepdims=True)
    masks = jnp.logical_or(idx == i1, idx == i2).astype(jnp.float32)
    weights = gates * masks / (m1 + m2 + 1e-9)
    out_ref[...] = jnp.concatenate(
        [weights[:, :E], masks[:, :E],
         jnp.zeros((V, EP - 2 * E), jnp.float32)], axis=1)
    wt_ref[...] = jnp.transpose(weights[:, :E], (1, 0))


def _router_table(router_w1, router_b1, router_w2, router_b2):
    w1 = router_w1.reshape(1, RH)
    b1 = router_b1.reshape(1, RH)
    w2 = jnp.zeros((RH, EP), jnp.float32).at[:, :E].set(router_w2)
    b2 = jnp.full((1, EP), -1e30, jnp.float32).at[0, :E].set(router_b2)
    return pl.pallas_call(
        _router_body,
        out_shape=[jax.ShapeDtypeStruct((V, EP), jnp.float32),
                   jax.ShapeDtypeStruct((E, V), jnp.float32)],
    )(w1, b1, w2, b2)


def _dense_body(xt_ref, wt_ref, ew_ref, eb_ref, cw_ref, cb_ref, out_ref,
                st_ref):
    a = ew_ref[...] * (1.0 / 783.0)
    c2b = (2.0 * jnp.cos(CH * a))[:, None, :]
    c2a = 2.0 * jnp.cos(a)
    rows = [jnp.sin(eb_ref[...]), jnp.sin(a + eb_ref[...])]
    for _ in range(2, 2 * CH):
        rows.append(c2a * rows[-1] - rows[-2])
    s0 = jnp.stack(rows[:CH], axis=1)
    s1 = jnp.stack(rows[CH:], axis=1)

    st_ref[0:1] = s0.astype(jnp.bfloat16).reshape(1, E * CH, D)
    st_ref[1:2] = s1.astype(jnp.bfloat16).reshape(1, E * CH, D)

    def body(j, carry):
        sp, sc = carry
        cur = c2b * sc - sp
        st_ref[pl.ds(j, 1)] = cur.astype(jnp.bfloat16).reshape(1, E * CH, D)
        return (sc, cur)

    lax.fori_loop(2, CL, body, (s0, s1))

    lane = lax.broadcasted_iota(jnp.int32, (1, LP), 1)
    ones_row = jnp.ones((1, S), jnp.bfloat16)
    rows = []
    for b in range(NB):
        oh = (xt_ref[:, b:b + 1] == lane).astype(jnp.bfloat16)
        rows.append(jnp.dot(ones_row, oh,
                            preferred_element_type=jnp.float32))
    cnt = jnp.concatenate(rows, axis=0)[:, :V]
    pooled = jnp.zeros((NB, D), jnp.float32)
    for e in range(E):
        lhs = (cnt * wt_ref[e:e + 1, :]).astype(jnp.bfloat16)
        se = st_ref[:, e * CH:(e + 1) * CH, :].reshape(V, D)
        pooled = pooled + jnp.dot(lhs, se,
                                  preferred_element_type=jnp.float32)
    out_ref[...] = jnp.dot(pooled * (1.0 / S), cw_ref[...],
                           preferred_element_type=jnp.float32) + cb_ref[...]


def _dense_logits(xt, wt, expert_w, expert_b, cls_w, cls_b):
    cw = jnp.zeros((D, EP), jnp.float32).at[:, :NC].set(cls_w)
    cb = jnp.zeros((1, EP), jnp.float32).at[0, :NC].set(cls_b)
    return pl.pallas_call(
        _dense_body,
        out_shape=jax.ShapeDtypeStruct((NB, EP), jnp.float32),
        scratch_shapes=[pltpu.VMEM((CL, E * CH, D), jnp.bfloat16)],
    )(xt, wt, expert_w, expert_b, cw, cb)


def _gather_tokens(wm_table, idx2d):
    info = plsc.get_sparse_core_info()
    nw = info.num_cores * info.num_subcores
    b_per_w = N // nw
    ch = 128
    nch = b_per_w // ch
    mesh = plsc.VectorSubcoreMesh(core_axis_name="c", subcore_axis_name="s")

    @functools.partial(
        pl.kernel, mesh=mesh,
        out_type=jax.ShapeDtypeStruct((N, EP), jnp.float32),
        scratch_types=[pltpu.VMEM((nch, ch), jnp.int32),
                       pltpu.VMEM((b_per_w, EP), jnp.float32),
                       pltpu.SemaphoreType.DMA],
    )
    def k(table_hbm, idx_hbm, out_hbm, idx_v, rows_v, sem):
        s = lax.axis_index("s")
        c = lax.axis_index("c")
        wid = s * info.num_cores + c
        pltpu.sync_copy(idx_hbm.at[pl.ds(wid * nch, nch)], idx_v)
        descs = [pltpu.async_copy(table_hbm.at[idx_v.at[j]],
                                  rows_v.at[pl.ds(j * ch, ch)], sem)
                 for j in range(nch)]
        for d in descs:
            d.wait()
        pltpu.sync_copy(rows_v, out_hbm.at[pl.ds(wid * b_per_w, b_per_w)])

    return k(wm_table, idx2d)


def kernel(x, expert_w, expert_b, router_w1, router_b1, router_w2,
           router_b2, cls_w, cls_b):
    xi = x.astype(jnp.int32)
    tab, wt = _router_table(router_w1, router_b1, router_w2, router_b2)
    wm = _gather_tokens(tab, xi.reshape(-1, 128))
    logits = _dense_logits(xi.T, wt, expert_w, expert_b, cls_w, cls_b)
    return logits[:, :NC], wm[:, :E], wm[:, E:2 * E]
